# Initial kernel scaffold; baseline (speedup 1.0000x reference)
import jax
import jax.numpy as jnp
from jax.experimental import pallas as pl


def kernel(user, positive, negative, user_table, item_table, graph_row, graph_col, graph_val):
    raise NotImplementedError("write your pallas kernel here")


# speedup vs baseline: 4.0508x; 4.0508x over previous
import functools

import jax
import jax.numpy as jnp
from jax import lax
from jax.experimental import pallas as pl
from jax.experimental.pallas import tpu as pltpu
from jax.experimental.pallas import tpu_sc as plsc

_NUM_USERS = 25000
_N_NODES = 50000
_N_EDGES = 800000
_D = 64
_LAYERS = 3
_GAMMA = 0.4
_C1 = (1.0 - _GAMMA) / 3.0
_REG = 1e-4
_BATCH = 4096

_NC = 2
_NS = 16
_HALF_E = _N_EDGES // 2
_HALF_N = _N_NODES // 2
_K = 128
_CHUNKS = _HALF_E // _K
_CPT = _CHUNKS // _NS
_CREM = _CHUNKS % _NS
_WB = 1560
_WREM = _HALF_N - _NS * _WB
_ZR = 104
_ZN = _WB // _ZR

_VK = 2000
_VCHUNKS = _N_EDGES // _VK
_VCPT = _VCHUNKS // (_NC * _NS)
_VCREM = _VCHUNKS % (_NC * _NS)

_NB = 3 * _BATCH
_BPT = _NB // (_NC * _NS)
_BK = 128
_BC = _BPT // _BK


def _mesh():
    return plsc.VectorSubcoreMesh(
        core_axis_name="c", subcore_axis_name="s",
        num_cores=_NC, num_subcores=_NS)


def _inv_norm_body(x_ref, o_ref):
    x = x_ref[...]
    ss = jnp.sum(x * x, axis=1, keepdims=True)
    o_ref[...] = 1.0 / (jnp.sqrt(ss) + 1e-12)


def _inv_norm(emb):
    rows = 2000
    out = pl.pallas_call(
        _inv_norm_body,
        grid=(_N_NODES // rows,),
        in_specs=[pl.BlockSpec((rows, _D), lambda i: (i, 0))],
        out_specs=pl.BlockSpec((rows, 1), lambda i: (i, 0)),
        out_shape=jax.ShapeDtypeStruct((_N_NODES, 1), jnp.float32),
    )(emb)
    return out.reshape(_N_NODES)


def _vescale_body(invn_h, col_h, val_h, ve_h, invn_v, col_v, val_v, ve_v):
    c = lax.axis_index("c")
    s = lax.axis_index("s")
    wid = s * _NC + c
    pltpu.sync_copy(invn_h, invn_v)
    start = wid * _VCPT + jnp.minimum(wid, _VCREM)
    cnt = _VCPT + jnp.where(wid < _VCREM, 1, 0)

    def chunk(ci, carry):
        e0 = ci * _VK
        pltpu.sync_copy(col_h.at[pl.ds(e0, _VK)], col_v)
        pltpu.sync_copy(val_h.at[pl.ds(e0, _VK)], val_v)

        def grp(g, carry2):
            sl = pl.ds(g * 16, 16)
            ve_v[sl] = plsc.load_gather(invn_v, [col_v[sl]]) * val_v[sl]
            return carry2
        lax.fori_loop(0, _VK // 16, grp, 0)
        pltpu.sync_copy(ve_v, ve_h.at[pl.ds(e0, _VK)])
        return carry
    lax.fori_loop(start, start + cnt, chunk, 0)


def _vescale(invn, cols, vals):
    f = pl.kernel(
        _vescale_body,
        out_type=jax.ShapeDtypeStruct((_N_EDGES,), jnp.float32),
        mesh=_mesh(),
        scratch_types=[
            pltpu.VMEM((_N_NODES,), jnp.float32),
            pltpu.VMEM((_VK,), jnp.int32),
            pltpu.VMEM((_VK,), jnp.float32),
            pltpu.VMEM((_VK,), jnp.float32),
        ],
        compiler_params=pltpu.CompilerParams(
            needs_layout_passes=False, use_tc_tiling_on_sc=False),
    )
    return f(invn, cols, vals)


def _spmm_body(emb_h, row_h, col_h, ve_h, out_h,
               acc, col_v, rraw_v, row_v, ve_v, gat_v, zero_v, sem):
    c = lax.axis_index("c")
    s = lax.axis_index("s")

    def zrow(i, carry):
        for d in range(_D // 16):
            zero_v[i, pl.ds(d * 16, 16)] = jnp.zeros((16,), jnp.float32)
        return carry
    lax.fori_loop(0, _ZR, zrow, 0)
    r0 = s * _WB

    def zcopy(j, carry):
        pltpu.sync_copy(zero_v, acc.at[pl.ds(r0 + j * _ZR, _ZR)])
        return carry
    lax.fori_loop(0, _ZN, zcopy, 0)

    @pl.when(s == 0)
    def _():
        pltpu.sync_copy(zero_v.at[pl.ds(0, _WREM)],
                        acc.at[pl.ds(_NS * _WB, _WREM)])
    plsc.subcore_barrier()

    base_n = c * _HALF_N
    start = s * _CPT + jnp.minimum(s, _CREM)
    cnt = _CPT + jnp.where(s < _CREM, 1, 0)

    def chunk(ci, carry):
        e0 = c * _HALF_E + ci * _K
        pltpu.sync_copy(col_h.at[pl.ds(e0, _K)], col_v)
        pltpu.sync_copy(row_h.at[pl.ds(e0, _K)], rraw_v)
        pltpu.sync_copy(ve_h.at[pl.ds(e0, _K)], ve_v)
        pltpu.async_copy(emb_h.at[col_v], gat_v, sem).wait()
        for kb in range(_K // 16):
            sl = pl.ds(kb * 16, 16)
            row_v[sl] = rraw_v[sl] - base_n

        def scale(k, carry2):
            sc = plsc.load_gather(
                ve_v, [jnp.full((16,), k, jnp.int32)])
            for d in range(_D // 16):
                sl2 = pl.ds(d * 16, 16)
                gat_v[k, sl2] = gat_v[k, sl2] * sc
            return carry2
        lax.fori_loop(0, _K, scale, 0)
        pltpu.sync_copy(gat_v, acc.at[row_v], add=True)
        return carry
    lax.fori_loop(start, start + cnt, chunk, 0)
    plsc.subcore_barrier()

    pltpu.sync_copy(acc.at[pl.ds(r0, _WB)],
                    out_h.at[pl.ds(base_n + r0, _WB)])

    @pl.when(s == 0)
    def _():
        pltpu.sync_copy(acc.at[pl.ds(_NS * _WB, _WREM)],
                        out_h.at[pl.ds(base_n + _NS * _WB, _WREM)])


def _spmm(emb, rows, cols, ve):
    f = pl.kernel(
        _spmm_body,
        out_type=jax.ShapeDtypeStruct((_N_NODES, _D), jnp.float32),
        mesh=_mesh(),
        scratch_types=[
            pltpu.VMEM_SHARED((_HALF_N, _D), jnp.float32),
            pltpu.VMEM((_K,), jnp.int32),
            pltpu.VMEM((_K,), jnp.int32),
            pltpu.VMEM((_K,), jnp.int32),
            pltpu.VMEM((_K,), jnp.float32),
            pltpu.VMEM((_K, _D), jnp.float32),
            pltpu.VMEM((_ZR, _D), jnp.float32),
            pltpu.SemaphoreType.DMA,
        ],
        compiler_params=pltpu.CompilerParams(
            needs_layout_passes=False, use_tc_tiling_on_sc=False),
    )
    return f(emb, rows, cols, ve)


def _combine_body(e0_h, e1_h, e2_h, e3_h, idx_h, fin_h, ego_h,
                  idx_v, g0, g1, g2, g3, fin_v, sem):
    c = lax.axis_index("c")
    s = lax.axis_index("s")
    wid = s * _NC + c

    def chunk(ci, carry):
        b0 = wid * _BPT + ci * _BK
        pltpu.sync_copy(idx_h.at[pl.ds(b0, _BK)], idx_v)
        pltpu.async_copy(e0_h.at[idx_v], g0, sem).wait()
        pltpu.async_copy(e1_h.at[idx_v], g1, sem).wait()
        pltpu.async_copy(e2_h.at[idx_v], g2, sem).wait()
        pltpu.async_copy(e3_h.at[idx_v], g3, sem).wait()

        def comb(k, carry2):
            for d in range(_D // 16):
                sl = pl.ds(d * 16, 16)
                fin_v[k, sl] = (_GAMMA * g0[k, sl]
                                + _C1 * (g1[k, sl] + g2[k, sl] + g3[k, sl]))
            return carry2
        lax.fori_loop(0, _BK, comb, 0)
        pltpu.sync_copy(fin_v, fin_h.at[pl.ds(b0, _BK)])
        pltpu.sync_copy(g0, ego_h.at[pl.ds(b0, _BK)])
        return carry
    lax.fori_loop(0, _BC, chunk, 0)


def _gather_combine(e0, e1, e2, e3, idx):
    f = pl.kernel(
        _combine_body,
        out_type=(jax.ShapeDtypeStruct((_NB, _D), jnp.float32),
                  jax.ShapeDtypeStruct((_NB, _D), jnp.float32)),
        mesh=_mesh(),
        scratch_types=[
            pltpu.VMEM((_BK,), jnp.int32),
            pltpu.VMEM((_BK, _D), jnp.float32),
            pltpu.VMEM((_BK, _D), jnp.float32),
            pltpu.VMEM((_BK, _D), jnp.float32),
            pltpu.VMEM((_BK, _D), jnp.float32),
            pltpu.VMEM((_BK, _D), jnp.float32),
            pltpu.SemaphoreType.DMA,
        ],
        compiler_params=pltpu.CompilerParams(
            needs_layout_passes=False, use_tc_tiling_on_sc=False),
    )
    return f(e0, e1, e2, e3, idx)


def _loss_body(fin_ref, ego_ref, bpr_ref, reg_ref):
    f = fin_ref[...]
    u = f[0:_BATCH]
    p = f[_BATCH:2 * _BATCH]
    n = f[2 * _BATCH:3 * _BATCH]
    pos = jnp.sum(u * p, axis=1)
    neg = jnp.sum(u * n, axis=1)
    bpr = jnp.mean(jax.nn.softplus(neg - pos))
    e = ego_ref[...]
    reg = (0.5 * _REG / _BATCH) * jnp.sum(e * e)
    bpr_ref[...] = jnp.reshape(bpr, (1, 1))
    reg_ref[...] = jnp.reshape(reg, (1, 1))


def _loss(fin, ego):
    return pl.pallas_call(
        _loss_body,
        out_shape=(jax.ShapeDtypeStruct((1, 1), jnp.float32),
                   jax.ShapeDtypeStruct((1, 1), jnp.float32)),
    )(fin, ego)


def kernel(user, positive, negative, user_table, item_table,
           graph_row, graph_col, graph_val):
    emb = jnp.concatenate([user_table, item_table], axis=0)
    embs = [emb]
    for _ in range(_LAYERS):
        invn = _inv_norm(embs[-1])
        ve = _vescale(invn, graph_col, graph_val)
        embs.append(_spmm(embs[-1], graph_row, graph_col, ve))
    idx = jnp.concatenate(
        [user, positive + _NUM_USERS, negative + _NUM_USERS])
    fin, ego = _gather_combine(embs[0], embs[1], embs[2], embs[3], idx)
    bpr, reg = _loss(fin, ego)
    return (bpr[0, 0], reg[0, 0])


# speedup vs baseline: 8.3037x; 2.0499x over previous
import functools

import jax
import jax.numpy as jnp
from jax import lax
from jax.experimental import pallas as pl
from jax.experimental.pallas import tpu as pltpu
from jax.experimental.pallas import tpu_sc as plsc

_NUM_USERS = 25000
_N_NODES = 50000
_N_EDGES = 800000
_D = 64
_LAYERS = 3
_GAMMA = 0.4
_C1 = (1.0 - _GAMMA) / 3.0
_REG = 1e-4
_BATCH = 4096

_NC = 2
_NS = 16
_HALF_E = _N_EDGES // 2
_HALF_N = _N_NODES // 2
_K = 128
_CHUNKS = _HALF_E // _K
_CPT = _CHUNKS // _NS
_CREM = _CHUNKS % _NS
_WB = 1560
_WREM = _HALF_N - _NS * _WB
_ZR = 40
_ZN = _WB // _ZR

_M = 25
_ME = _M * _K
_BLKS = _CHUNKS // _M
_BPTS = _BLKS // _NS
_BREM = _BLKS % _NS

_NB = 3 * _BATCH
_BPT = _NB // (_NC * _NS)
_BK = 128
_BC = _BPT // _BK


def _mesh():
    return plsc.VectorSubcoreMesh(
        core_axis_name="c", subcore_axis_name="s",
        num_cores=_NC, num_subcores=_NS)


def _normalize_body(x_ref, o_ref):
    x = x_ref[...]
    ss = jnp.sum(x * x, axis=1, keepdims=True)
    o_ref[...] = x * (1.0 / (jnp.sqrt(ss) + 1e-12))


def _normalize(emb):
    rows = 2000
    return pl.pallas_call(
        _normalize_body,
        grid=(_N_NODES // rows,),
        in_specs=[pl.BlockSpec((rows, _D), lambda i: (i, 0))],
        out_specs=pl.BlockSpec((rows, _D), lambda i: (i, 0)),
        out_shape=jax.ShapeDtypeStruct((_N_NODES, _D), jnp.float32),
    )(emb)


def _spmm_body(emb_h, row_h, col_h, val_h, out_h,
               acc, colm_v, rowm_v, valm_v,
               col0_v, col1_v, row0_v, row1_v,
               gat0_v, gat1_v, zero_v,
               semg0, semg1, sems0, sems1):
    c = lax.axis_index("c")
    s = lax.axis_index("s")
    col_b = (col0_v, col1_v)
    row_b = (row0_v, row1_v)
    gat_b = (gat0_v, gat1_v)
    semg = (semg0, semg1)
    sems = (sems0, sems1)

    def zrow(i, carry):
        for d in range(_D // 16):
            zero_v[i, pl.ds(d * 16, 16)] = jnp.zeros((16,), jnp.float32)
        return carry
    lax.fori_loop(0, _ZR, zrow, 0)
    r0 = s * _WB

    def zcopy(j, carry):
        pltpu.sync_copy(zero_v, acc.at[pl.ds(r0 + j * _ZR, _ZR)])
        return carry
    lax.fori_loop(0, _ZN, zcopy, 0)

    @pl.when(s == 0)
    def _():
        pltpu.sync_copy(zero_v, acc.at[pl.ds(_NS * _WB, _WREM)])
    plsc.subcore_barrier()

    base_n = c * _HALF_N
    start = s * _BPTS + jnp.minimum(s, _BREM)
    cnt = _BPTS + jnp.where(s < _BREM, 1, 0)

    def build_col(j, b):
        def grp(g, carry):
            sl = pl.ds(g * 16, 16)
            col_b[b][sl] = colm_v[pl.ds(j * _K + g * 16, 16)]
            return carry
        lax.fori_loop(0, _K // 16, grp, 0)

    def block(bi, carry):
        e0 = c * _HALF_E + bi * _ME
        pltpu.sync_copy(col_h.at[pl.ds(e0, _ME)], colm_v)
        pltpu.sync_copy(row_h.at[pl.ds(e0, _ME)], rowm_v)
        pltpu.sync_copy(val_h.at[pl.ds(e0, _ME)], valm_v)
        build_col(0, 0)
        pltpu.async_copy(emb_h.at[col0_v], gat0_v, semg0)

        def chunk(j, carry2):
            def halfstep(b):
                nb = 1 - b

                @pl.when(j < _M - 1)
                def _():
                    @pl.when(j >= 1)
                    def _():
                        pltpu.make_async_copy(
                            gat_b[nb], acc.at[row_b[nb]], sems[nb]).wait()
                    build_col(jnp.int32(j) + 1, nb)
                    pltpu.async_copy(
                        emb_h.at[col_b[nb]], gat_b[nb], semg[nb])
                pltpu.make_async_copy(
                    emb_h.at[col_b[b]], gat_b[b], semg[b]).wait()

                def grp(g, carry3):
                    sl = pl.ds(g * 16, 16)
                    row_b[b][sl] = rowm_v[pl.ds(j * _K + g * 16, 16)] - base_n
                    return carry3
                lax.fori_loop(0, _K // 16, grp, 0)

                def scale(k, carry3):
                    sc = plsc.load_gather(
                        valm_v, [jnp.full((16,), j * _K + k, jnp.int32)])
                    for d in range(_D // 16):
                        sl2 = pl.ds(d * 16, 16)
                        gat_b[b][k, sl2] = gat_b[b][k, sl2] * sc
                    return carry3
                lax.fori_loop(0, _K, scale, 0)
                pltpu.make_async_copy(
                    gat_b[b], acc.at[row_b[b]], sems[b]).start(add=True)

            @pl.when(j % 2 == 0)
            def _():
                halfstep(0)

            @pl.when(j % 2 == 1)
            def _():
                halfstep(1)
            return carry2
        lax.fori_loop(0, _M, chunk, 0)
        pltpu.make_async_copy(gat0_v, acc.at[row0_v], sems0).wait()
        pltpu.make_async_copy(gat1_v, acc.at[row1_v], sems1).wait()
        return carry
    lax.fori_loop(start, start + cnt, block, 0)
    plsc.subcore_barrier()

    pltpu.sync_copy(acc.at[pl.ds(r0, _WB)],
                    out_h.at[pl.ds(base_n + r0, _WB)])

    @pl.when(s == 0)
    def _():
        pltpu.sync_copy(acc.at[pl.ds(_NS * _WB, _WREM)],
                        out_h.at[pl.ds(base_n + _NS * _WB, _WREM)])


def _spmm(emb, rows, cols, vals):
    f = pl.kernel(
        _spmm_body,
        out_type=jax.ShapeDtypeStruct((_N_NODES, _D), jnp.float32),
        mesh=_mesh(),
        scratch_types=[
            pltpu.VMEM_SHARED((_HALF_N, _D), jnp.float32),
            pltpu.VMEM((_ME,), jnp.int32),
            pltpu.VMEM((_ME,), jnp.int32),
            pltpu.VMEM((_ME,), jnp.float32),
            pltpu.VMEM((_K,), jnp.int32),
            pltpu.VMEM((_K,), jnp.int32),
            pltpu.VMEM((_K,), jnp.int32),
            pltpu.VMEM((_K,), jnp.int32),
            pltpu.VMEM((_K, _D), jnp.float32),
            pltpu.VMEM((_K, _D), jnp.float32),
            pltpu.VMEM((_ZR, _D), jnp.float32),
            pltpu.SemaphoreType.DMA,
            pltpu.SemaphoreType.DMA,
            pltpu.SemaphoreType.DMA,
            pltpu.SemaphoreType.DMA,
        ],
        compiler_params=pltpu.CompilerParams(
            needs_layout_passes=False, use_tc_tiling_on_sc=False),
    )
    return f(emb, rows, cols, vals)


def _combine_body(e0_h, e1_h, e2_h, e3_h, idx_h, fin_h, ego_h,
                  idx_v, g0, g1, g2, g3, fin_v, sem):
    c = lax.axis_index("c")
    s = lax.axis_index("s")
    wid = s * _NC + c

    def chunk(ci, carry):
        b0 = wid * _BPT + ci * _BK
        pltpu.sync_copy(idx_h.at[pl.ds(b0, _BK)], idx_v)
        pltpu.async_copy(e0_h.at[idx_v], g0, sem).wait()
        pltpu.async_copy(e1_h.at[idx_v], g1, sem).wait()
        pltpu.async_copy(e2_h.at[idx_v], g2, sem).wait()
        pltpu.async_copy(e3_h.at[idx_v], g3, sem).wait()

        def comb(k, carry2):
            for d in range(_D // 16):
                sl = pl.ds(d * 16, 16)
                fin_v[k, sl] = (_GAMMA * g0[k, sl]
                                + _C1 * (g1[k, sl] + g2[k, sl] + g3[k, sl]))
            return carry2
        lax.fori_loop(0, _BK, comb, 0)
        pltpu.sync_copy(fin_v, fin_h.at[pl.ds(b0, _BK)])
        pltpu.sync_copy(g0, ego_h.at[pl.ds(b0, _BK)])
        return carry
    lax.fori_loop(0, _BC, chunk, 0)


def _gather_combine(e0, e1, e2, e3, idx):
    f = pl.kernel(
        _combine_body,
        out_type=(jax.ShapeDtypeStruct((_NB, _D), jnp.float32),
                  jax.ShapeDtypeStruct((_NB, _D), jnp.float32)),
        mesh=_mesh(),
        scratch_types=[
            pltpu.VMEM((_BK,), jnp.int32),
            pltpu.VMEM((_BK, _D), jnp.float32),
            pltpu.VMEM((_BK, _D), jnp.float32),
            pltpu.VMEM((_BK, _D), jnp.float32),
            pltpu.VMEM((_BK, _D), jnp.float32),
            pltpu.VMEM((_BK, _D), jnp.float32),
            pltpu.SemaphoreType.DMA,
        ],
        compiler_params=pltpu.CompilerParams(
            needs_layout_passes=False, use_tc_tiling_on_sc=False),
    )
    return f(e0, e1, e2, e3, idx)


def _loss_body(fin_ref, ego_ref, bpr_ref, reg_ref):
    f = fin_ref[...]
    u = f[0:_BATCH]
    p = f[_BATCH:2 * _BATCH]
    n = f[2 * _BATCH:3 * _BATCH]
    pos = jnp.sum(u * p, axis=1)
    neg = jnp.sum(u * n, axis=1)
    bpr = jnp.mean(jax.nn.softplus(neg - pos))
    e = ego_ref[...]
    reg = (0.5 * _REG / _BATCH) * jnp.sum(e * e)
    bpr_ref[...] = jnp.reshape(bpr, (1, 1))
    reg_ref[...] = jnp.reshape(reg, (1, 1))


def _loss(fin, ego):
    return pl.pallas_call(
        _loss_body,
        out_shape=(jax.ShapeDtypeStruct((1, 1), jnp.float32),
                   jax.ShapeDtypeStruct((1, 1), jnp.float32)),
    )(fin, ego)


def kernel(user, positive, negative, user_table, item_table,
           graph_row, graph_col, graph_val):
    emb = jnp.concatenate([user_table, item_table], axis=0)
    embs = [emb]
    for _ in range(_LAYERS):
        emb_n = _normalize(embs[-1])
        embs.append(_spmm(emb_n, graph_row, graph_col, graph_val))
    idx = jnp.concatenate(
        [user, positive + _NUM_USERS, negative + _NUM_USERS])
    fin, ego = _gather_combine(embs[0], embs[1], embs[2], embs[3], idx)
    bpr, reg = _loss(fin, ego)
    return (bpr[0, 0], reg[0, 0])


# speedup vs baseline: 11.2755x; 1.3579x over previous
import functools

import jax
import jax.numpy as jnp
from jax import lax
from jax.experimental import pallas as pl
from jax.experimental.pallas import tpu as pltpu
from jax.experimental.pallas import tpu_sc as plsc

_NUM_USERS = 25000
_N_NODES = 50000
_N_EDGES = 800000
_D = 64
_LAYERS = 3
_GAMMA = 0.4
_C1 = (1.0 - _GAMMA) / 3.0
_REG = 1e-4
_BATCH = 4096

_NC = 2
_NS = 16
_HALF_E = _N_EDGES // 2
_HALF_N = _N_NODES // 2
_K = 128
_CHUNKS = _HALF_E // _K
_CPT = _CHUNKS // _NS
_CREM = _CHUNKS % _NS
_WB = 1560
_WREM = _HALF_N - _NS * _WB
_ZR = 40
_ZN = _WB // _ZR

_M = 25
_ME = _M * _K
_BLKS = _CHUNKS // _M
_BPTS = _BLKS // _NS
_BREM = _BLKS % _NS

_NB = 3 * _BATCH
_BPT = _NB // (_NC * _NS)
_BK = 128
_BC = _BPT // _BK


def _mesh():
    return plsc.VectorSubcoreMesh(
        core_axis_name="c", subcore_axis_name="s",
        num_cores=_NC, num_subcores=_NS)


def _normalize_body(x_ref, o_ref):
    x = x_ref[...]
    ss = jnp.sum(x * x, axis=1, keepdims=True)
    o_ref[...] = x * (1.0 / (jnp.sqrt(ss) + 1e-12))


def _normalize(emb):
    rows = 2000
    return pl.pallas_call(
        _normalize_body,
        grid=(_N_NODES // rows,),
        in_specs=[pl.BlockSpec((rows, _D), lambda i: (i, 0))],
        out_specs=pl.BlockSpec((rows, _D), lambda i: (i, 0)),
        out_shape=jax.ShapeDtypeStruct((_N_NODES, _D), jnp.float32),
    )(emb)


def _spmm_body(emb_h, row_h, col_h, val_h, out_h,
               acc, colm_v, rowm_v, valm_v,
               col0_v, col1_v, row0_v, row1_v,
               gat0_v, gat1_v, zero_v,
               semg0, semg1, sems0, sems1):
    c = lax.axis_index("c")
    s = lax.axis_index("s")
    col_b = (col0_v, col1_v)
    row_b = (row0_v, row1_v)
    gat_b = (gat0_v, gat1_v)
    semg = (semg0, semg1)
    sems = (sems0, sems1)

    def zrow(i, carry):
        for d in range(_D // 16):
            zero_v[i, pl.ds(d * 16, 16)] = jnp.zeros((16,), jnp.float32)
        return carry
    lax.fori_loop(0, _ZR, zrow, 0)
    r0 = s * _WB

    def zcopy(j, carry):
        pltpu.sync_copy(zero_v, acc.at[pl.ds(r0 + j * _ZR, _ZR)])
        return carry
    lax.fori_loop(0, _ZN, zcopy, 0)

    @pl.when(s == 0)
    def _():
        pltpu.sync_copy(zero_v, acc.at[pl.ds(_NS * _WB, _WREM)])
    plsc.subcore_barrier()

    base_n = c * _HALF_N
    start = s * _BPTS + jnp.minimum(s, _BREM)
    cnt = _BPTS + jnp.where(s < _BREM, 1, 0)

    def build_col(j, b):
        @plsc.parallel_loop(0, _K // 16, unroll=4)
        def _(g):
            sl = pl.ds(g * 16, 16)
            col_b[b][sl] = colm_v[pl.ds(j * _K + g * 16, 16)]

    def block(bi, carry):
        e0 = c * _HALF_E + bi * _ME
        pltpu.sync_copy(col_h.at[pl.ds(e0, _ME)], colm_v)
        pltpu.sync_copy(row_h.at[pl.ds(e0, _ME)], rowm_v)
        pltpu.sync_copy(val_h.at[pl.ds(e0, _ME)], valm_v)
        build_col(0, 0)
        pltpu.async_copy(emb_h.at[col0_v], gat0_v, semg0)

        def chunk(j, carry2):
            def halfstep(b):
                nb = 1 - b

                @pl.when(j < _M - 1)
                def _():
                    @pl.when(j >= 1)
                    def _():
                        pltpu.make_async_copy(
                            gat_b[nb], acc.at[row_b[nb]], sems[nb]).wait()
                    build_col(jnp.int32(j) + 1, nb)
                    pltpu.async_copy(
                        emb_h.at[col_b[nb]], gat_b[nb], semg[nb])
                pltpu.make_async_copy(
                    emb_h.at[col_b[b]], gat_b[b], semg[b]).wait()

                @plsc.parallel_loop(0, _K // 16, unroll=4)
                def _(g):
                    sl = pl.ds(g * 16, 16)
                    row_b[b][sl] = rowm_v[pl.ds(j * _K + g * 16, 16)] - base_n

                @plsc.parallel_loop(0, _K, unroll=4)
                def _(k):
                    sc = plsc.load_gather(
                        valm_v, [jnp.full((16,), j * _K + k, jnp.int32)])
                    for d in range(_D // 16):
                        sl2 = pl.ds(d * 16, 16)
                        gat_b[b][k, sl2] = gat_b[b][k, sl2] * sc
                pltpu.make_async_copy(
                    gat_b[b], acc.at[row_b[b]], sems[b]).start(add=True)

            @pl.when(j % 2 == 0)
            def _():
                halfstep(0)

            @pl.when(j % 2 == 1)
            def _():
                halfstep(1)
            return carry2
        lax.fori_loop(0, _M, chunk, 0)
        pltpu.make_async_copy(gat0_v, acc.at[row0_v], sems0).wait()
        pltpu.make_async_copy(gat1_v, acc.at[row1_v], sems1).wait()
        return carry
    lax.fori_loop(start, start + cnt, block, 0)
    plsc.subcore_barrier()

    pltpu.sync_copy(acc.at[pl.ds(r0, _WB)],
                    out_h.at[pl.ds(base_n + r0, _WB)])

    @pl.when(s == 0)
    def _():
        pltpu.sync_copy(acc.at[pl.ds(_NS * _WB, _WREM)],
                        out_h.at[pl.ds(base_n + _NS * _WB, _WREM)])


def _spmm(emb, rows, cols, vals):
    f = pl.kernel(
        _spmm_body,
        out_type=jax.ShapeDtypeStruct((_N_NODES, _D), jnp.float32),
        mesh=_mesh(),
        scratch_types=[
            pltpu.VMEM_SHARED((_HALF_N, _D), jnp.float32),
            pltpu.VMEM((_ME,), jnp.int32),
            pltpu.VMEM((_ME,), jnp.int32),
            pltpu.VMEM((_ME,), jnp.float32),
            pltpu.VMEM((_K,), jnp.int32),
            pltpu.VMEM((_K,), jnp.int32),
            pltpu.VMEM((_K,), jnp.int32),
            pltpu.VMEM((_K,), jnp.int32),
            pltpu.VMEM((_K, _D), jnp.float32),
            pltpu.VMEM((_K, _D), jnp.float32),
            pltpu.VMEM((_ZR, _D), jnp.float32),
            pltpu.SemaphoreType.DMA,
            pltpu.SemaphoreType.DMA,
            pltpu.SemaphoreType.DMA,
            pltpu.SemaphoreType.DMA,
        ],
        compiler_params=pltpu.CompilerParams(
            needs_layout_passes=False, use_tc_tiling_on_sc=False),
    )
    return f(emb, rows, cols, vals)


def _combine_body(e0_h, e1_h, e2_h, e3_h, idx_h, fin_h, ego_h,
                  idx_v, g0, g1, g2, g3, fin_v, sem):
    c = lax.axis_index("c")
    s = lax.axis_index("s")
    wid = s * _NC + c

    def chunk(ci, carry):
        b0 = wid * _BPT + ci * _BK
        pltpu.sync_copy(idx_h.at[pl.ds(b0, _BK)], idx_v)
        pltpu.async_copy(e0_h.at[idx_v], g0, sem).wait()
        pltpu.async_copy(e1_h.at[idx_v], g1, sem).wait()
        pltpu.async_copy(e2_h.at[idx_v], g2, sem).wait()
        pltpu.async_copy(e3_h.at[idx_v], g3, sem).wait()

        def comb(k, carry2):
            for d in range(_D // 16):
                sl = pl.ds(d * 16, 16)
                fin_v[k, sl] = (_GAMMA * g0[k, sl]
                                + _C1 * (g1[k, sl] + g2[k, sl] + g3[k, sl]))
            return carry2
        lax.fori_loop(0, _BK, comb, 0)
        pltpu.sync_copy(fin_v, fin_h.at[pl.ds(b0, _BK)])
        pltpu.sync_copy(g0, ego_h.at[pl.ds(b0, _BK)])
        return carry
    lax.fori_loop(0, _BC, chunk, 0)


def _gather_combine(e0, e1, e2, e3, idx):
    f = pl.kernel(
        _combine_body,
        out_type=(jax.ShapeDtypeStruct((_NB, _D), jnp.float32),
                  jax.ShapeDtypeStruct((_NB, _D), jnp.float32)),
        mesh=_mesh(),
        scratch_types=[
            pltpu.VMEM((_BK,), jnp.int32),
            pltpu.VMEM((_BK, _D), jnp.float32),
            pltpu.VMEM((_BK, _D), jnp.float32),
            pltpu.VMEM((_BK, _D), jnp.float32),
            pltpu.VMEM((_BK, _D), jnp.float32),
            pltpu.VMEM((_BK, _D), jnp.float32),
            pltpu.SemaphoreType.DMA,
        ],
        compiler_params=pltpu.CompilerParams(
            needs_layout_passes=False, use_tc_tiling_on_sc=False),
    )
    return f(e0, e1, e2, e3, idx)


def _loss_body(fin_ref, ego_ref, bpr_ref, reg_ref):
    f = fin_ref[...]
    u = f[0:_BATCH]
    p = f[_BATCH:2 * _BATCH]
    n = f[2 * _BATCH:3 * _BATCH]
    pos = jnp.sum(u * p, axis=1)
    neg = jnp.sum(u * n, axis=1)
    bpr = jnp.mean(jax.nn.softplus(neg - pos))
    e = ego_ref[...]
    reg = (0.5 * _REG / _BATCH) * jnp.sum(e * e)
    bpr_ref[...] = jnp.reshape(bpr, (1, 1))
    reg_ref[...] = jnp.reshape(reg, (1, 1))


def _loss(fin, ego):
    return pl.pallas_call(
        _loss_body,
        out_shape=(jax.ShapeDtypeStruct((1, 1), jnp.float32),
                   jax.ShapeDtypeStruct((1, 1), jnp.float32)),
    )(fin, ego)


def kernel(user, positive, negative, user_table, item_table,
           graph_row, graph_col, graph_val):
    emb = jnp.concatenate([user_table, item_table], axis=0)
    embs = [emb]
    for _ in range(_LAYERS):
        emb_n = _normalize(embs[-1])
        embs.append(_spmm(emb_n, graph_row, graph_col, graph_val))
    idx = jnp.concatenate(
        [user, positive + _NUM_USERS, negative + _NUM_USERS])
    fin, ego = _gather_combine(embs[0], embs[1], embs[2], embs[3], idx)
    bpr, reg = _loss(fin, ego)
    return (bpr[0, 0], reg[0, 0])
